# baseline (speedup 1.0000x reference)
import functools

import jax
import jax.numpy as jnp
from jax import lax
from jax.experimental import pallas as pl
from jax.experimental.pallas import tpu as pltpu
from jax.experimental.pallas import tpu_sc as plsc

NUM_ANGLES = 100000
EMBED_DIM = 32
PAD_DIM = 128
BATCH = 16384
HIST = 200

B = BATCH * HIST
NC = 2
NS = 16
NW = NC * NS
IDX_MINOR = 200
LISTS_PER_ROW = HIST // IDX_MINOR
B_PER_GROUP = 4
ROWS_PER_GROUP = B_PER_GROUP * HIST
K = B_PER_GROUP * LISTS_PER_ROW
NBUF = 4
B_PER_BODY = NBUF * B_PER_GROUP
B_PER_WORKER = BATCH // NW
BODIES = B_PER_WORKER // B_PER_BODY


def _make_sc_gather():
    mesh = plsc.VectorSubcoreMesh(core_axis_name="c", subcore_axis_name="s")

    @functools.partial(
        pl.kernel,
        mesh=mesh,
        out_type=jax.ShapeDtypeStruct((B, PAD_DIM), jnp.float32),
        scratch_types=[
            pltpu.VMEM((B_PER_BODY, HIST), jnp.int32),
            [pltpu.VMEM((ROWS_PER_GROUP, EMBED_DIM), jnp.float32)
             for _ in range(NBUF)],
            [pltpu.SemaphoreType.DMA for _ in range(NBUF)],
            [pltpu.SemaphoreType.DMA for _ in range(NBUF)],
        ],
        compiler_params=pltpu.CompilerParams(use_tc_tiling_on_sc=False),
    )
    def gather_kernel(theta_hbm, table_hbm, out_hbm,
                      idx_v, rows, gsem, ssem):
        wid = lax.axis_index("s") * NC + lax.axis_index("c")
        row_base = wid * B_PER_WORKER

        def out_slice(t, q):
            flat = (row_base + t * B_PER_BODY + q * B_PER_GROUP) * HIST
            return out_hbm.at[pl.ds(flat, ROWS_PER_GROUP),
                              pl.ds(0, EMBED_DIM)]

        def load_idx(t):
            pltpu.sync_copy(
                theta_hbm.at[pl.ds(row_base + t * B_PER_BODY, B_PER_BODY)],
                idx_v)

        def fire(q):
            return [
                pltpu.async_copy(
                    table_hbm.at[idx_v.at[q * B_PER_GROUP + j]],
                    rows[q].at[pl.ds(j * IDX_MINOR, IDX_MINOR)],
                    gsem[q],
                )
                for j in range(K)
            ]

        load_idx(0)
        handles = [fire(q) for q in range(NBUF)]
        for q in range(NBUF):
            for h in handles[q]:
                h.wait()
            pltpu.async_copy(rows[q], out_slice(0, q), ssem[q])

        def body(t, _):
            load_idx(t)
            hs = []
            for q in range(NBUF):
                pltpu.make_async_copy(rows[q], out_slice(t, q),
                                      ssem[q]).wait()
                hs.append(fire(q))
            for q in range(NBUF):
                for h in hs[q]:
                    h.wait()
                pltpu.async_copy(rows[q], out_slice(t, q), ssem[q])
            return ()

        lax.fori_loop(1, BODIES, body, (), unroll=False)

        for q in range(NBUF):
            pltpu.make_async_copy(rows[q], out_slice(0, q), ssem[q]).wait()

    return gather_kernel


_sc_gather = _make_sc_gather()


def kernel(theta, table):
    out_pad = _sc_gather(theta.astype(jnp.int32), table)
    return out_pad.reshape(BATCH, HIST, PAD_DIM)[:, :, :EMBED_DIM]


# speedup vs baseline: 1.0269x; 1.0269x over previous
import functools

import jax
import jax.numpy as jnp
from jax import lax
from jax.experimental import pallas as pl
from jax.experimental.pallas import tpu as pltpu
from jax.experimental.pallas import tpu_sc as plsc

NUM_ANGLES = 100000
EMBED_DIM = 32
PAD_DIM = 128
BATCH = 16384
HIST = 200

B = BATCH * HIST
NC = 2
NS = 16
NW = NC * NS
IDX_MINOR = 128
ROWS_PER_GROUP = 1024
K = ROWS_PER_GROUP // IDX_MINOR
GROUPS = B // (NW * ROWS_PER_GROUP)
IDX_ROWS_PER_WORKER = B // (NW * IDX_MINOR)


def _make_sc_gather():
    mesh = plsc.VectorSubcoreMesh(core_axis_name="c", subcore_axis_name="s")

    @functools.partial(
        pl.kernel,
        mesh=mesh,
        out_type=jax.ShapeDtypeStruct((B, PAD_DIM), jnp.float32),
        scratch_types=[
            pltpu.VMEM((K, IDX_MINOR), jnp.int32),
            pltpu.VMEM((K, IDX_MINOR), jnp.int32),
            pltpu.VMEM((ROWS_PER_GROUP, EMBED_DIM), jnp.float32),
            pltpu.VMEM((ROWS_PER_GROUP, EMBED_DIM), jnp.float32),
            pltpu.SemaphoreType.DMA,
            pltpu.SemaphoreType.DMA,
            pltpu.SemaphoreType.DMA,
            pltpu.SemaphoreType.DMA,
            pltpu.SemaphoreType.DMA,
            pltpu.SemaphoreType.DMA,
        ],
        compiler_params=pltpu.CompilerParams(use_tc_tiling_on_sc=False),
    )
    def gather_kernel(idx_hbm, table_hbm, out_hbm,
                      idx0, idx1, rows0, rows1,
                      gsem0, gsem1, ssem0, ssem1, isem0, isem1):
        wid = lax.axis_index("s") * NC + lax.axis_index("c")
        idx_row_base = wid * IDX_ROWS_PER_WORKER
        out_base = wid * IDX_ROWS_PER_WORKER * IDX_MINOR

        def out_slice(g):
            return out_hbm.at[pl.ds(out_base + g * ROWS_PER_GROUP,
                                    ROWS_PER_GROUP), pl.ds(0, EMBED_DIM)]

        def idx_src(g):
            return idx_hbm.at[pl.ds(idx_row_base + g * K, K)]

        def fire(idx_v, rows_v, gsem):
            return [
                pltpu.async_copy(
                    table_hbm.at[idx_v.at[j]],
                    rows_v.at[pl.ds(j * IDX_MINOR, IDX_MINOR)],
                    gsem,
                )
                for j in range(K)
            ]

        i0 = pltpu.async_copy(idx_src(0), idx0, isem0)
        i1 = pltpu.async_copy(idx_src(1), idx1, isem1)
        i0.wait()
        h0 = fire(idx0, rows0, gsem0)
        i1.wait()
        h1 = fire(idx1, rows1, gsem1)
        for h in h0:
            h.wait()
        pltpu.async_copy(rows0, out_slice(0), ssem0)
        for h in h1:
            h.wait()
        pltpu.async_copy(rows1, out_slice(1), ssem1)
        pltpu.async_copy(idx_src(2), idx0, isem0)
        pltpu.async_copy(idx_src(3), idx1, isem1)

        def body(gg, _):
            a = 2 * gg
            b = a + 1
            pltpu.make_async_copy(idx_src(a), idx0, isem0).wait()
            pltpu.make_async_copy(idx_src(b), idx1, isem1).wait()
            pltpu.make_async_copy(rows0, out_slice(a), ssem0).wait()
            ha = fire(idx0, rows0, gsem0)
            pltpu.make_async_copy(rows1, out_slice(b), ssem1).wait()
            hb = fire(idx1, rows1, gsem1)
            for h in ha:
                h.wait()
            pltpu.async_copy(rows0, out_slice(a), ssem0)
            for h in hb:
                h.wait()
            pltpu.async_copy(rows1, out_slice(b), ssem1)
            nxt = jnp.minimum(gg + 1, GROUPS // 2 - 1)
            pltpu.async_copy(idx_src(2 * nxt), idx0, isem0)
            pltpu.async_copy(idx_src(2 * nxt + 1), idx1, isem1)
            return ()

        lax.fori_loop(1, GROUPS // 2, body, (), unroll=False)

        pltpu.make_async_copy(idx_src(0), idx0, isem0).wait()
        pltpu.make_async_copy(idx_src(1), idx1, isem1).wait()
        pltpu.make_async_copy(rows0, out_slice(0), ssem0).wait()
        pltpu.make_async_copy(rows1, out_slice(1), ssem1).wait()

    return gather_kernel


_sc_gather = _make_sc_gather()


def kernel(theta, table):
    idx2d = theta.reshape(B // IDX_MINOR, IDX_MINOR).astype(jnp.int32)
    out_pad = _sc_gather(idx2d, table)
    return out_pad.reshape(BATCH, HIST, PAD_DIM)[:, :, :EMBED_DIM]


# speedup vs baseline: 1.0309x; 1.0039x over previous
import functools

import jax
import jax.numpy as jnp
from jax import lax
from jax.experimental import pallas as pl
from jax.experimental.pallas import tpu as pltpu
from jax.experimental.pallas import tpu_sc as plsc

NUM_ANGLES = 100000
EMBED_DIM = 32
PAD_DIM = 128
BATCH = 16384
HIST = 200

B = BATCH * HIST
NC = 2
NS = 16
NW = NC * NS
IDX_MINOR = 128
ROWS_PER_GROUP = 1280
K = ROWS_PER_GROUP // IDX_MINOR
GROUPS = B // (NW * ROWS_PER_GROUP)
IDX_ROWS_PER_WORKER = B // (NW * IDX_MINOR)


def _make_sc_gather():
    mesh = plsc.VectorSubcoreMesh(core_axis_name="c", subcore_axis_name="s")

    @functools.partial(
        pl.kernel,
        mesh=mesh,
        out_type=jax.ShapeDtypeStruct((B, PAD_DIM), jnp.float32),
        scratch_types=[
            pltpu.VMEM((K, IDX_MINOR), jnp.int32),
            pltpu.VMEM((K, IDX_MINOR), jnp.int32),
            pltpu.VMEM((ROWS_PER_GROUP, EMBED_DIM), jnp.float32),
            pltpu.VMEM((ROWS_PER_GROUP, EMBED_DIM), jnp.float32),
            pltpu.SemaphoreType.DMA,
            pltpu.SemaphoreType.DMA,
            pltpu.SemaphoreType.DMA,
            pltpu.SemaphoreType.DMA,
        ],
        compiler_params=pltpu.CompilerParams(use_tc_tiling_on_sc=False),
    )
    def gather_kernel(idx_hbm, table_hbm, out_hbm,
                      idx0, idx1, rows0, rows1,
                      gsem0, gsem1, ssem0, ssem1):
        wid = lax.axis_index("s") * NC + lax.axis_index("c")
        idx_row_base = wid * IDX_ROWS_PER_WORKER
        out_base = wid * IDX_ROWS_PER_WORKER * IDX_MINOR

        def out_slice(g):
            return out_hbm.at[pl.ds(out_base + g * ROWS_PER_GROUP,
                                    ROWS_PER_GROUP), pl.ds(0, EMBED_DIM)]

        def load_and_fire(g, idx_v, rows_v, gsem):
            pltpu.sync_copy(idx_hbm.at[pl.ds(idx_row_base + g * K, K)], idx_v)
            return [
                pltpu.async_copy(
                    table_hbm.at[idx_v.at[j]],
                    rows_v.at[pl.ds(j * IDX_MINOR, IDX_MINOR)],
                    gsem,
                )
                for j in range(K)
            ]

        h0 = load_and_fire(0, idx0, rows0, gsem0)
        h1 = load_and_fire(1, idx1, rows1, gsem1)
        for h in h0:
            h.wait()
        pltpu.async_copy(rows0, out_slice(0), ssem0)
        for h in h1:
            h.wait()
        pltpu.async_copy(rows1, out_slice(1), ssem1)

        def body(gg, _):
            a = 2 * gg
            b = a + 1
            pltpu.make_async_copy(rows0, out_slice(a), ssem0).wait()
            ha = load_and_fire(a, idx0, rows0, gsem0)
            pltpu.make_async_copy(rows1, out_slice(b), ssem1).wait()
            hb = load_and_fire(b, idx1, rows1, gsem1)
            for h in ha:
                h.wait()
            pltpu.async_copy(rows0, out_slice(a), ssem0)
            for h in hb:
                h.wait()
            pltpu.async_copy(rows1, out_slice(b), ssem1)
            return ()

        lax.fori_loop(1, GROUPS // 2, body, (), unroll=False)

        pltpu.make_async_copy(rows0, out_slice(0), ssem0).wait()
        pltpu.make_async_copy(rows1, out_slice(1), ssem1).wait()

    return gather_kernel


_sc_gather = _make_sc_gather()


def kernel(theta, table):
    idx2d = theta.reshape(B // IDX_MINOR, IDX_MINOR).astype(jnp.int32)
    out_pad = _sc_gather(idx2d, table)
    return out_pad.reshape(BATCH, HIST, PAD_DIM)[:, :, :EMBED_DIM]


# speedup vs baseline: 1.0359x; 1.0049x over previous
import functools

import jax
import jax.numpy as jnp
from jax import lax
from jax.experimental import pallas as pl
from jax.experimental.pallas import tpu as pltpu
from jax.experimental.pallas import tpu_sc as plsc

NUM_ANGLES = 100000
EMBED_DIM = 32
PAD_DIM = 128
BATCH = 16384
HIST = 200

B = BATCH * HIST
NC = 2
NS = 16
NW = NC * NS
IDX_MINOR = 160
ROWS_PER_GROUP = 1600
K = ROWS_PER_GROUP // IDX_MINOR
GROUPS = B // (NW * ROWS_PER_GROUP)
IDX_ROWS_PER_WORKER = B // (NW * IDX_MINOR)


def _make_sc_gather():
    mesh = plsc.VectorSubcoreMesh(core_axis_name="c", subcore_axis_name="s")

    @functools.partial(
        pl.kernel,
        mesh=mesh,
        out_type=jax.ShapeDtypeStruct((B, PAD_DIM), jnp.float32),
        scratch_types=[
            pltpu.VMEM((K, IDX_MINOR), jnp.int32),
            pltpu.VMEM((K, IDX_MINOR), jnp.int32),
            pltpu.VMEM((ROWS_PER_GROUP, EMBED_DIM), jnp.float32),
            pltpu.VMEM((ROWS_PER_GROUP, EMBED_DIM), jnp.float32),
            pltpu.SemaphoreType.DMA,
            pltpu.SemaphoreType.DMA,
            pltpu.SemaphoreType.DMA,
            pltpu.SemaphoreType.DMA,
        ],
        compiler_params=pltpu.CompilerParams(use_tc_tiling_on_sc=False),
    )
    def gather_kernel(idx_hbm, table_hbm, out_hbm,
                      idx0, idx1, rows0, rows1,
                      gsem0, gsem1, ssem0, ssem1):
        wid = lax.axis_index("s") * NC + lax.axis_index("c")
        idx_row_base = wid * IDX_ROWS_PER_WORKER
        out_base = wid * IDX_ROWS_PER_WORKER * IDX_MINOR

        def out_slice(g):
            return out_hbm.at[pl.ds(out_base + g * ROWS_PER_GROUP,
                                    ROWS_PER_GROUP), pl.ds(0, EMBED_DIM)]

        def load_and_fire(g, idx_v, rows_v, gsem):
            pltpu.sync_copy(idx_hbm.at[pl.ds(idx_row_base + g * K, K)], idx_v)
            return [
                pltpu.async_copy(
                    table_hbm.at[idx_v.at[j]],
                    rows_v.at[pl.ds(j * IDX_MINOR, IDX_MINOR)],
                    gsem,
                )
                for j in range(K)
            ]

        h0 = load_and_fire(0, idx0, rows0, gsem0)
        h1 = load_and_fire(1, idx1, rows1, gsem1)
        for h in h0:
            h.wait()
        pltpu.async_copy(rows0, out_slice(0), ssem0)
        for h in h1:
            h.wait()
        pltpu.async_copy(rows1, out_slice(1), ssem1)

        def body(gg, _):
            a = 2 * gg
            b = a + 1
            pltpu.make_async_copy(rows0, out_slice(a), ssem0).wait()
            ha = load_and_fire(a, idx0, rows0, gsem0)
            pltpu.make_async_copy(rows1, out_slice(b), ssem1).wait()
            hb = load_and_fire(b, idx1, rows1, gsem1)
            for h in ha:
                h.wait()
            pltpu.async_copy(rows0, out_slice(a), ssem0)
            for h in hb:
                h.wait()
            pltpu.async_copy(rows1, out_slice(b), ssem1)
            return ()

        lax.fori_loop(1, GROUPS // 2, body, (), unroll=False)

        pltpu.make_async_copy(rows0, out_slice(0), ssem0).wait()
        pltpu.make_async_copy(rows1, out_slice(1), ssem1).wait()

    return gather_kernel


_sc_gather = _make_sc_gather()


def kernel(theta, table):
    idx2d = theta.reshape(B // IDX_MINOR, IDX_MINOR).astype(jnp.int32)
    out_pad = _sc_gather(idx2d, table)
    return out_pad.reshape(BATCH, HIST, PAD_DIM)[:, :, :EMBED_DIM]
